# Initial kernel scaffold; baseline (speedup 1.0000x reference)
import jax
import jax.numpy as jnp
from jax.experimental import pallas as pl


def kernel(s0, s1, edge_index, drop_prob, x_0, training, W_pre, bn_gamma, bn_beta, bn_mean, bn_var, W_gin):
    raise NotImplementedError("write your pallas kernel here")


# speedup vs baseline: 24.3163x; 24.3163x over previous
import functools
import math

import jax
import jax.numpy as jnp
from jax import lax
from jax.experimental import pallas as pl
from jax.experimental.pallas import tpu as pltpu
from jax.experimental.pallas import tpu_sc as plsc

N = 10000
E = 320000
D = 128
ALPHA = 0.1
BETA = float(math.log(0.5 / 1 + 1.0))

NPAD = 10240
ROWS_PER_TILE = NPAD // 16
EW = 125
EROWS = E // EW
WORKERS = 32
ROWS_PER_WORKER = EROWS // WORKERS
CH = 8
NCH = ROWS_PER_WORKER // CH

_MESH = plsc.VectorSubcoreMesh(core_axis_name="c", subcore_axis_name="s")


@functools.partial(
    pl.kernel,
    mesh=_MESH,
    out_type=jax.ShapeDtypeStruct((2, NPAD, D), jnp.float32),
    scratch_types=[
        pltpu.VMEM((CH, EW), jnp.int32),
        pltpu.VMEM((EW, D), jnp.float32),
        pltpu.VMEM_SHARED((NPAD, D), jnp.float32),
    ],
)
def _deg_call(dst2, ones_hbm, zeros_hbm, out, dstbuf, onesbuf, acc):
    c = lax.axis_index("c")
    s = lax.axis_index("s")
    wid = c * 16 + s
    r0 = s * ROWS_PER_TILE
    pltpu.sync_copy(zeros_hbm.at[pl.ds(r0, ROWS_PER_TILE)],
                    acc.at[pl.ds(r0, ROWS_PER_TILE)])
    pltpu.sync_copy(ones_hbm, onesbuf)
    plsc.subcore_barrier()
    erow0 = wid * ROWS_PER_WORKER

    def body(ch, carry):
        base = erow0 + ch * CH
        pltpu.sync_copy(dst2.at[pl.ds(base, CH)], dstbuf)
        for j in range(CH):
            pltpu.sync_copy(onesbuf, acc.at[dstbuf.at[j]], add=True)
        return carry

    lax.fori_loop(0, NCH, body, 0)
    plsc.subcore_barrier()
    pltpu.sync_copy(acc.at[pl.ds(r0, ROWS_PER_TILE)],
                    out.at[c, pl.ds(r0, ROWS_PER_TILE)])


@functools.partial(
    pl.kernel,
    mesh=_MESH,
    out_type=jax.ShapeDtypeStruct((2, NPAD, D), jnp.float32),
    scratch_types=[
        pltpu.VMEM((CH, EW), jnp.int32),
        pltpu.VMEM((CH, EW), jnp.int32),
        pltpu.VMEM((EW, D), jnp.float32),
        pltpu.VMEM_SHARED((NPAD, D), jnp.float32),
    ],
)
def _agg_call(src2, dst2, hs_hbm, zeros_hbm, out, srcbuf, dstbuf, rowbuf, acc):
    c = lax.axis_index("c")
    s = lax.axis_index("s")
    wid = c * 16 + s
    r0 = s * ROWS_PER_TILE
    pltpu.sync_copy(zeros_hbm.at[pl.ds(r0, ROWS_PER_TILE)],
                    acc.at[pl.ds(r0, ROWS_PER_TILE)])
    plsc.subcore_barrier()
    erow0 = wid * ROWS_PER_WORKER

    def body(ch, carry):
        base = erow0 + ch * CH
        pltpu.sync_copy(src2.at[pl.ds(base, CH)], srcbuf)
        pltpu.sync_copy(dst2.at[pl.ds(base, CH)], dstbuf)
        for j in range(CH):
            pltpu.sync_copy(hs_hbm.at[srcbuf.at[j]], rowbuf)
            pltpu.sync_copy(rowbuf, acc.at[dstbuf.at[j]], add=True)
        return carry

    lax.fori_loop(0, NCH, body, 0)
    plsc.subcore_barrier()
    pltpu.sync_copy(acc.at[pl.ds(r0, ROWS_PER_TILE)],
                    out.at[c, pl.ds(r0, ROWS_PER_TILE)])


BN_ROWS = 2000


def _pre_body(s1_ref, w_ref, g_ref, b_ref, m_ref, v_ref, h_ref):
    h = jnp.dot(s1_ref[...], w_ref[...], preferred_element_type=jnp.float32)
    h = (h - m_ref[...]) * (g_ref[...] * lax.rsqrt(v_ref[...] + 1e-5)) + b_ref[...]
    h_ref[...] = jnp.maximum(h, 0.0)


def _scale_body(h_ref, dp_ref, hs_ref):
    deg = dp_ref[0, :, 0] + dp_ref[1, :, 0] + 1.0
    dinv = lax.rsqrt(jnp.maximum(deg, 1.0))
    hs_ref[...] = h_ref[...] * dinv[:, None]


def _out_body(acc_ref, hs_ref, x0_ref, dp_ref, wg_ref, o_ref):
    deg = dp_ref[0, :, 0] + dp_ref[1, :, 0] + 1.0
    dinv = lax.rsqrt(jnp.maximum(deg, 1.0))
    accsum = acc_ref[0] + acc_ref[1] + hs_ref[...]
    support = (1.0 - ALPHA) * (accsum * dinv[:, None]) + ALPHA * x0_ref[...]
    o_ref[...] = jnp.maximum(
        (1.0 - BETA) * support
        + BETA * jnp.dot(support, wg_ref[...], preferred_element_type=jnp.float32),
        0.0,
    )


def _pre_call(s1, W_pre, bn_gamma, bn_beta, bn_mean, bn_var):
    grid = (N // BN_ROWS,)
    vec = pl.BlockSpec((1, D), lambda i: (0, 0))
    return pl.pallas_call(
        _pre_body,
        grid=grid,
        in_specs=[
            pl.BlockSpec((BN_ROWS, D), lambda i: (i, 0)),
            pl.BlockSpec((D, D), lambda i: (0, 0)),
            vec, vec, vec, vec,
        ],
        out_specs=pl.BlockSpec((BN_ROWS, D), lambda i: (i, 0)),
        out_shape=jax.ShapeDtypeStruct((N, D), jnp.float32),
    )(s1, W_pre, bn_gamma.reshape(1, D), bn_beta.reshape(1, D),
      bn_mean.reshape(1, D), bn_var.reshape(1, D))


def _scale_call(h, degparts):
    grid = (N // BN_ROWS,)
    return pl.pallas_call(
        _scale_body,
        grid=grid,
        in_specs=[
            pl.BlockSpec((BN_ROWS, D), lambda i: (i, 0)),
            pl.BlockSpec((2, BN_ROWS, D), lambda i: (0, i, 0)),
        ],
        out_specs=pl.BlockSpec((BN_ROWS, D), lambda i: (i, 0)),
        out_shape=jax.ShapeDtypeStruct((N, D), jnp.float32),
    )(h, degparts)


def _out_call(accparts, hs, x_0, degparts, W_gin):
    grid = (N // BN_ROWS,)
    return pl.pallas_call(
        _out_body,
        grid=grid,
        in_specs=[
            pl.BlockSpec((2, BN_ROWS, D), lambda i: (0, i, 0)),
            pl.BlockSpec((BN_ROWS, D), lambda i: (i, 0)),
            pl.BlockSpec((BN_ROWS, D), lambda i: (i, 0)),
            pl.BlockSpec((2, BN_ROWS, D), lambda i: (0, i, 0)),
            pl.BlockSpec((D, D), lambda i: (0, 0)),
        ],
        out_specs=pl.BlockSpec((BN_ROWS, D), lambda i: (i, 0)),
        out_shape=jax.ShapeDtypeStruct((N, D), jnp.float32),
    )(accparts, hs, x_0, degparts, W_gin)


def kernel(s0, s1, edge_index, drop_prob, x_0, training,
           W_pre, bn_gamma, bn_beta, bn_mean, bn_var, W_gin):
    src2 = edge_index[0].astype(jnp.int32).reshape(EROWS, EW)
    dst2 = edge_index[1].astype(jnp.int32).reshape(EROWS, EW)
    ones_ew = jnp.ones((EW, D), jnp.float32)
    zeros_acc = jnp.zeros((NPAD, D), jnp.float32)

    degparts = _deg_call(dst2, ones_ew, zeros_acc)
    h = _pre_call(s1, W_pre, bn_gamma, bn_beta, bn_mean, bn_var)
    hs = _scale_call(h, degparts)
    accparts = _agg_call(src2, dst2, hs, zeros_acc)
    return _out_call(accparts, hs, x_0, degparts, W_gin)


# speedup vs baseline: 27.9100x; 1.1478x over previous
import functools
import math

import jax
import jax.numpy as jnp
from jax import lax
from jax.experimental import pallas as pl
from jax.experimental.pallas import tpu as pltpu
from jax.experimental.pallas import tpu_sc as plsc

N = 10000
E = 320000
D = 128
ALPHA = 0.1
BETA = float(math.log(0.5 / 1 + 1.0))

NPAD = 10240
ROWS_PER_TILE = NPAD // 16
EW = 125
EROWS = E // EW
WORKERS = 32
ROWS_PER_WORKER = EROWS // WORKERS
CH = 8
NCH = ROWS_PER_WORKER // CH

_MESH = plsc.VectorSubcoreMesh(core_axis_name="c", subcore_axis_name="s")


@functools.partial(
    pl.kernel,
    mesh=_MESH,
    out_type=jax.ShapeDtypeStruct((2, NPAD, D), jnp.float32),
    scratch_types=[
        pltpu.VMEM((CH, EW), jnp.int32),
        pltpu.VMEM((EW, D), jnp.float32),
        pltpu.VMEM_SHARED((NPAD, D), jnp.float32),
    ],
)
def _deg_call(dst2, ones_hbm, zeros_hbm, out, dstbuf, onesbuf, acc):
    c = lax.axis_index("c")
    s = lax.axis_index("s")
    wid = c * 16 + s
    r0 = s * ROWS_PER_TILE
    pltpu.sync_copy(zeros_hbm.at[pl.ds(r0, ROWS_PER_TILE)],
                    acc.at[pl.ds(r0, ROWS_PER_TILE)])
    pltpu.sync_copy(ones_hbm, onesbuf)
    plsc.subcore_barrier()
    erow0 = wid * ROWS_PER_WORKER

    def body(ch, carry):
        base = erow0 + ch * CH
        pltpu.sync_copy(dst2.at[pl.ds(base, CH)], dstbuf)
        for j in range(CH):
            pltpu.sync_copy(onesbuf, acc.at[dstbuf.at[j]], add=True)
        return carry

    lax.fori_loop(0, NCH, body, 0)
    plsc.subcore_barrier()
    pltpu.sync_copy(acc.at[pl.ds(r0, ROWS_PER_TILE)],
                    out.at[c, pl.ds(r0, ROWS_PER_TILE)])


@functools.partial(
    pl.kernel,
    mesh=_MESH,
    out_type=jax.ShapeDtypeStruct((2, NPAD, D), jnp.float32),
    scratch_types=[
        pltpu.VMEM((CH, EW), jnp.int32),
        pltpu.VMEM((CH, EW), jnp.int32),
        pltpu.VMEM((EW, D), jnp.float32),
        pltpu.VMEM((EW, D), jnp.float32),
        pltpu.VMEM_SHARED((NPAD, D), jnp.float32),
        pltpu.SemaphoreType.DMA,
        pltpu.SemaphoreType.DMA,
        pltpu.SemaphoreType.DMA,
        pltpu.SemaphoreType.DMA,
    ],
)
def _agg_call(src2, dst2, hs_hbm, zeros_hbm, out,
              srcbuf, dstbuf, rowbuf0, rowbuf1, acc, gs0, gs1, ss0, ss1):
    c = lax.axis_index("c")
    s = lax.axis_index("s")
    wid = c * 16 + s
    r0 = s * ROWS_PER_TILE
    pltpu.sync_copy(zeros_hbm.at[pl.ds(r0, ROWS_PER_TILE)],
                    acc.at[pl.ds(r0, ROWS_PER_TILE)])
    plsc.subcore_barrier()
    erow0 = wid * ROWS_PER_WORKER
    bufs = (rowbuf0, rowbuf1)
    gsems = (gs0, gs1)
    ssems = (ss0, ss1)

    def body(ch, carry):
        base = erow0 + ch * CH
        pltpu.sync_copy(src2.at[pl.ds(base, CH)], srcbuf)
        pltpu.sync_copy(dst2.at[pl.ds(base, CH)], dstbuf)
        gh = [None, None]
        sh = [None, None]
        gh[0] = pltpu.async_copy(hs_hbm.at[srcbuf.at[0]], bufs[0], gsems[0])
        for j in range(CH):
            b = j % 2
            gh[b].wait()
            if j + 1 < CH:
                nb = (j + 1) % 2
                if sh[nb] is not None:
                    sh[nb].wait()
                gh[nb] = pltpu.async_copy(
                    hs_hbm.at[srcbuf.at[j + 1]], bufs[nb], gsems[nb])
            sh[b] = pltpu.async_copy(
                bufs[b], acc.at[dstbuf.at[j]], ssems[b], add=True)
        sh[0].wait()
        sh[1].wait()
        return carry

    lax.fori_loop(0, NCH, body, 0)
    plsc.subcore_barrier()
    pltpu.sync_copy(acc.at[pl.ds(r0, ROWS_PER_TILE)],
                    out.at[c, pl.ds(r0, ROWS_PER_TILE)])


BN_ROWS = 2000


def _pre_body(s1_ref, w_ref, g_ref, b_ref, m_ref, v_ref, h_ref):
    h = jnp.dot(s1_ref[...], w_ref[...], preferred_element_type=jnp.float32)
    h = (h - m_ref[...]) * (g_ref[...] * lax.rsqrt(v_ref[...] + 1e-5)) + b_ref[...]
    h_ref[...] = jnp.maximum(h, 0.0)


def _scale_body(h_ref, dp_ref, hs_ref):
    deg = dp_ref[0, :, 0] + dp_ref[1, :, 0] + 1.0
    dinv = lax.rsqrt(jnp.maximum(deg, 1.0))
    hs_ref[...] = h_ref[...] * dinv[:, None]


def _out_body(acc_ref, hs_ref, x0_ref, dp_ref, wg_ref, o_ref):
    deg = dp_ref[0, :, 0] + dp_ref[1, :, 0] + 1.0
    dinv = lax.rsqrt(jnp.maximum(deg, 1.0))
    accsum = acc_ref[0] + acc_ref[1] + hs_ref[...]
    support = (1.0 - ALPHA) * (accsum * dinv[:, None]) + ALPHA * x0_ref[...]
    o_ref[...] = jnp.maximum(
        (1.0 - BETA) * support
        + BETA * jnp.dot(support, wg_ref[...], preferred_element_type=jnp.float32),
        0.0,
    )


def _pre_call(s1, W_pre, bn_gamma, bn_beta, bn_mean, bn_var):
    grid = (N // BN_ROWS,)
    vec = pl.BlockSpec((1, D), lambda i: (0, 0))
    return pl.pallas_call(
        _pre_body,
        grid=grid,
        in_specs=[
            pl.BlockSpec((BN_ROWS, D), lambda i: (i, 0)),
            pl.BlockSpec((D, D), lambda i: (0, 0)),
            vec, vec, vec, vec,
        ],
        out_specs=pl.BlockSpec((BN_ROWS, D), lambda i: (i, 0)),
        out_shape=jax.ShapeDtypeStruct((N, D), jnp.float32),
    )(s1, W_pre, bn_gamma.reshape(1, D), bn_beta.reshape(1, D),
      bn_mean.reshape(1, D), bn_var.reshape(1, D))


def _scale_call(h, degparts):
    grid = (N // BN_ROWS,)
    return pl.pallas_call(
        _scale_body,
        grid=grid,
        in_specs=[
            pl.BlockSpec((BN_ROWS, D), lambda i: (i, 0)),
            pl.BlockSpec((2, BN_ROWS, D), lambda i: (0, i, 0)),
        ],
        out_specs=pl.BlockSpec((BN_ROWS, D), lambda i: (i, 0)),
        out_shape=jax.ShapeDtypeStruct((N, D), jnp.float32),
    )(h, degparts)


def _out_call(accparts, hs, x_0, degparts, W_gin):
    grid = (N // BN_ROWS,)
    return pl.pallas_call(
        _out_body,
        grid=grid,
        in_specs=[
            pl.BlockSpec((2, BN_ROWS, D), lambda i: (0, i, 0)),
            pl.BlockSpec((BN_ROWS, D), lambda i: (i, 0)),
            pl.BlockSpec((BN_ROWS, D), lambda i: (i, 0)),
            pl.BlockSpec((2, BN_ROWS, D), lambda i: (0, i, 0)),
            pl.BlockSpec((D, D), lambda i: (0, 0)),
        ],
        out_specs=pl.BlockSpec((BN_ROWS, D), lambda i: (i, 0)),
        out_shape=jax.ShapeDtypeStruct((N, D), jnp.float32),
    )(accparts, hs, x_0, degparts, W_gin)


def kernel(s0, s1, edge_index, drop_prob, x_0, training,
           W_pre, bn_gamma, bn_beta, bn_mean, bn_var, W_gin):
    src2 = edge_index[0].astype(jnp.int32).reshape(EROWS, EW)
    dst2 = edge_index[1].astype(jnp.int32).reshape(EROWS, EW)
    ones_ew = jnp.ones((EW, D), jnp.float32)
    zeros_acc = jnp.zeros((NPAD, D), jnp.float32)

    degparts = _deg_call(dst2, ones_ew, zeros_acc)
    h = _pre_call(s1, W_pre, bn_gamma, bn_beta, bn_mean, bn_var)
    hs = _scale_call(h, degparts)
    accparts = _agg_call(src2, dst2, hs, zeros_acc)
    return _out_call(accparts, hs, x_0, degparts, W_gin)


# speedup vs baseline: 28.3906x; 1.0172x over previous
import functools
import math

import jax
import jax.numpy as jnp
from jax import lax
from jax.experimental import pallas as pl
from jax.experimental.pallas import tpu as pltpu
from jax.experimental.pallas import tpu_sc as plsc

N = 10000
E = 320000
D = 128
ALPHA = 0.1
BETA = float(math.log(0.5 / 1 + 1.0))

NPAD = 10240
ROWS_PER_TILE = NPAD // 16
EW = 125
EROWS = E // EW
WORKERS = 32
ROWS_PER_WORKER = EROWS // WORKERS
CH = 8
NCH = ROWS_PER_WORKER // CH

_MESH = plsc.VectorSubcoreMesh(core_axis_name="c", subcore_axis_name="s")


@functools.partial(
    pl.kernel,
    mesh=_MESH,
    out_type=jax.ShapeDtypeStruct((2, NPAD, D), jnp.float32),
    scratch_types=[
        pltpu.VMEM((ROWS_PER_WORKER, EW), jnp.int32),
        pltpu.VMEM((EW, D), jnp.float32),
        pltpu.VMEM_SHARED((NPAD, D), jnp.float32),
        pltpu.SemaphoreType.DMA,
    ],
)
def _deg_call(dst2, ones_hbm, zeros_hbm, out, dstbuf, onesbuf, acc, dsem):
    c = lax.axis_index("c")
    s = lax.axis_index("s")
    wid = c * 16 + s
    r0 = s * ROWS_PER_TILE
    pltpu.sync_copy(zeros_hbm.at[pl.ds(r0, ROWS_PER_TILE)],
                    acc.at[pl.ds(r0, ROWS_PER_TILE)])
    pltpu.sync_copy(ones_hbm, onesbuf)
    erow0 = wid * ROWS_PER_WORKER
    pltpu.sync_copy(dst2.at[pl.ds(erow0, ROWS_PER_WORKER)], dstbuf)
    plsc.subcore_barrier()

    def body(ch, carry):
        hs_ = [pltpu.async_copy(onesbuf, acc.at[dstbuf.at[ch * CH + j]],
                                dsem, add=True)
               for j in range(CH)]
        for h_ in hs_:
            h_.wait()
        return carry

    lax.fori_loop(0, NCH, body, 0)
    plsc.subcore_barrier()
    pltpu.sync_copy(acc.at[pl.ds(r0, ROWS_PER_TILE)],
                    out.at[c, pl.ds(r0, ROWS_PER_TILE)])


@functools.partial(
    pl.kernel,
    mesh=_MESH,
    out_type=jax.ShapeDtypeStruct((2, NPAD, D), jnp.float32),
    scratch_types=[
        pltpu.VMEM((CH, EW), jnp.int32),
        pltpu.VMEM((CH, EW), jnp.int32),
        pltpu.VMEM((EW, D), jnp.float32),
        pltpu.VMEM((EW, D), jnp.float32),
        pltpu.VMEM_SHARED((NPAD, D), jnp.float32),
        pltpu.SemaphoreType.DMA,
        pltpu.SemaphoreType.DMA,
        pltpu.SemaphoreType.DMA,
        pltpu.SemaphoreType.DMA,
    ],
)
def _agg_call(src2, dst2, hs_hbm, zeros_hbm, out,
              srcbuf, dstbuf, rowbuf0, rowbuf1, acc, gs0, gs1, ss0, ss1):
    c = lax.axis_index("c")
    s = lax.axis_index("s")
    wid = c * 16 + s
    r0 = s * ROWS_PER_TILE
    pltpu.sync_copy(zeros_hbm.at[pl.ds(r0, ROWS_PER_TILE)],
                    acc.at[pl.ds(r0, ROWS_PER_TILE)])
    plsc.subcore_barrier()
    erow0 = wid * ROWS_PER_WORKER
    bufs = (rowbuf0, rowbuf1)
    gsems = (gs0, gs1)
    ssems = (ss0, ss1)

    def body(ch, carry):
        base = erow0 + ch * CH
        pltpu.sync_copy(src2.at[pl.ds(base, CH)], srcbuf)
        pltpu.sync_copy(dst2.at[pl.ds(base, CH)], dstbuf)
        gh = [None, None]
        sh = [None, None]
        gh[0] = pltpu.async_copy(hs_hbm.at[srcbuf.at[0]], bufs[0], gsems[0])
        for j in range(CH):
            b = j % 2
            gh[b].wait()
            if j + 1 < CH:
                nb = (j + 1) % 2
                if sh[nb] is not None:
                    sh[nb].wait()
                gh[nb] = pltpu.async_copy(
                    hs_hbm.at[srcbuf.at[j + 1]], bufs[nb], gsems[nb])
            sh[b] = pltpu.async_copy(
                bufs[b], acc.at[dstbuf.at[j]], ssems[b], add=True)
        sh[0].wait()
        sh[1].wait()
        return carry

    lax.fori_loop(0, NCH, body, 0)
    plsc.subcore_barrier()
    pltpu.sync_copy(acc.at[pl.ds(r0, ROWS_PER_TILE)],
                    out.at[c, pl.ds(r0, ROWS_PER_TILE)])


BN_ROWS = 2000


def _pre_body(s1_ref, w_ref, g_ref, b_ref, m_ref, v_ref, h_ref):
    h = jnp.dot(s1_ref[...], w_ref[...], preferred_element_type=jnp.float32)
    h = (h - m_ref[...]) * (g_ref[...] * lax.rsqrt(v_ref[...] + 1e-5)) + b_ref[...]
    h_ref[...] = jnp.maximum(h, 0.0)


def _scale_body(h_ref, dp_ref, hs_ref):
    deg = dp_ref[0, :, 0] + dp_ref[1, :, 0] + 1.0
    dinv = lax.rsqrt(jnp.maximum(deg, 1.0))
    hs_ref[...] = h_ref[...] * dinv[:, None]


def _out_body(acc_ref, hs_ref, x0_ref, dp_ref, wg_ref, o_ref):
    deg = dp_ref[0, :, 0] + dp_ref[1, :, 0] + 1.0
    dinv = lax.rsqrt(jnp.maximum(deg, 1.0))
    accsum = acc_ref[0] + acc_ref[1] + hs_ref[...]
    support = (1.0 - ALPHA) * (accsum * dinv[:, None]) + ALPHA * x0_ref[...]
    o_ref[...] = jnp.maximum(
        (1.0 - BETA) * support
        + BETA * jnp.dot(support, wg_ref[...], preferred_element_type=jnp.float32),
        0.0,
    )


def _pre_call(s1, W_pre, bn_gamma, bn_beta, bn_mean, bn_var):
    grid = (N // BN_ROWS,)
    vec = pl.BlockSpec((1, D), lambda i: (0, 0))
    return pl.pallas_call(
        _pre_body,
        grid=grid,
        in_specs=[
            pl.BlockSpec((BN_ROWS, D), lambda i: (i, 0)),
            pl.BlockSpec((D, D), lambda i: (0, 0)),
            vec, vec, vec, vec,
        ],
        out_specs=pl.BlockSpec((BN_ROWS, D), lambda i: (i, 0)),
        out_shape=jax.ShapeDtypeStruct((N, D), jnp.float32),
    )(s1, W_pre, bn_gamma.reshape(1, D), bn_beta.reshape(1, D),
      bn_mean.reshape(1, D), bn_var.reshape(1, D))


def _scale_call(h, degparts):
    grid = (N // BN_ROWS,)
    return pl.pallas_call(
        _scale_body,
        grid=grid,
        in_specs=[
            pl.BlockSpec((BN_ROWS, D), lambda i: (i, 0)),
            pl.BlockSpec((2, BN_ROWS, D), lambda i: (0, i, 0)),
        ],
        out_specs=pl.BlockSpec((BN_ROWS, D), lambda i: (i, 0)),
        out_shape=jax.ShapeDtypeStruct((N, D), jnp.float32),
    )(h, degparts)


def _out_call(accparts, hs, x_0, degparts, W_gin):
    grid = (N // BN_ROWS,)
    return pl.pallas_call(
        _out_body,
        grid=grid,
        in_specs=[
            pl.BlockSpec((2, BN_ROWS, D), lambda i: (0, i, 0)),
            pl.BlockSpec((BN_ROWS, D), lambda i: (i, 0)),
            pl.BlockSpec((BN_ROWS, D), lambda i: (i, 0)),
            pl.BlockSpec((2, BN_ROWS, D), lambda i: (0, i, 0)),
            pl.BlockSpec((D, D), lambda i: (0, 0)),
        ],
        out_specs=pl.BlockSpec((BN_ROWS, D), lambda i: (i, 0)),
        out_shape=jax.ShapeDtypeStruct((N, D), jnp.float32),
    )(accparts, hs, x_0, degparts, W_gin)


def kernel(s0, s1, edge_index, drop_prob, x_0, training,
           W_pre, bn_gamma, bn_beta, bn_mean, bn_var, W_gin):
    src2 = edge_index[0].astype(jnp.int32).reshape(EROWS, EW)
    dst2 = edge_index[1].astype(jnp.int32).reshape(EROWS, EW)
    ones_ew = jnp.ones((EW, D), jnp.float32)
    zeros_acc = jnp.zeros((NPAD, D), jnp.float32)

    degparts = _deg_call(dst2, ones_ew, zeros_acc)
    h = _pre_call(s1, W_pre, bn_gamma, bn_beta, bn_mean, bn_var)
    hs = _scale_call(h, degparts)
    accparts = _agg_call(src2, dst2, hs, zeros_acc)
    return _out_call(accparts, hs, x_0, degparts, W_gin)


# speedup vs baseline: 29.1283x; 1.0260x over previous
import functools
import math

import jax
import jax.numpy as jnp
from jax import lax
from jax.experimental import pallas as pl
from jax.experimental.pallas import tpu as pltpu
from jax.experimental.pallas import tpu_sc as plsc

N = 10000
E = 320000
D = 128
ALPHA = 0.1
BETA = float(math.log(0.5 / 1 + 1.0))

NPAD = 10240
ROWS_PER_TILE = NPAD // 16
EW = 125
EROWS = E // EW
WORKERS = 32
ROWS_PER_WORKER = EROWS // WORKERS
CH = 8
NCH = ROWS_PER_WORKER // CH

_MESH = plsc.VectorSubcoreMesh(core_axis_name="c", subcore_axis_name="s")


@functools.partial(
    pl.kernel,
    mesh=_MESH,
    out_type=jax.ShapeDtypeStruct((2, NPAD, D), jnp.float32),
    scratch_types=[
        pltpu.VMEM((ROWS_PER_WORKER, EW), jnp.int32),
        pltpu.VMEM((EW, D), jnp.float32),
        pltpu.VMEM_SHARED((NPAD, D), jnp.float32),
        pltpu.SemaphoreType.DMA,
    ],
)
def _deg_call(ei3, ones_hbm, zeros_hbm, out, dstbuf, onesbuf, acc, dsem):
    c = lax.axis_index("c")
    s = lax.axis_index("s")
    wid = c * 16 + s
    r0 = s * ROWS_PER_TILE
    pltpu.sync_copy(zeros_hbm.at[pl.ds(r0, ROWS_PER_TILE)],
                    acc.at[pl.ds(r0, ROWS_PER_TILE)])
    pltpu.sync_copy(ones_hbm, onesbuf)
    erow0 = wid * ROWS_PER_WORKER
    pltpu.sync_copy(ei3.at[1, pl.ds(erow0, ROWS_PER_WORKER)], dstbuf)
    plsc.subcore_barrier()

    def body(ch, carry):
        hs_ = [pltpu.async_copy(onesbuf, acc.at[dstbuf.at[ch * CH + j]],
                                dsem, add=True)
               for j in range(CH)]
        for h_ in hs_:
            h_.wait()
        return carry

    lax.fori_loop(0, NCH, body, 0)
    plsc.subcore_barrier()
    pltpu.sync_copy(acc.at[pl.ds(r0, ROWS_PER_TILE)],
                    out.at[c, pl.ds(r0, ROWS_PER_TILE)])


@functools.partial(
    pl.kernel,
    mesh=_MESH,
    out_type=jax.ShapeDtypeStruct((2, NPAD, D), jnp.float32),
    scratch_types=[
        pltpu.VMEM((CH, EW), jnp.int32),
        pltpu.VMEM((CH, EW), jnp.int32),
        pltpu.VMEM((EW, D), jnp.float32),
        pltpu.VMEM((EW, D), jnp.float32),
        pltpu.VMEM_SHARED((NPAD, D), jnp.float32),
        pltpu.SemaphoreType.DMA,
        pltpu.SemaphoreType.DMA,
        pltpu.SemaphoreType.DMA,
        pltpu.SemaphoreType.DMA,
    ],
)
def _agg_call(ei3, hs_hbm, zeros_hbm, out,
              srcbuf, dstbuf, rowbuf0, rowbuf1, acc, gs0, gs1, ss0, ss1):
    c = lax.axis_index("c")
    s = lax.axis_index("s")
    wid = c * 16 + s
    r0 = s * ROWS_PER_TILE
    pltpu.sync_copy(zeros_hbm.at[pl.ds(r0, ROWS_PER_TILE)],
                    acc.at[pl.ds(r0, ROWS_PER_TILE)])
    plsc.subcore_barrier()
    erow0 = wid * ROWS_PER_WORKER
    bufs = (rowbuf0, rowbuf1)
    gsems = (gs0, gs1)
    ssems = (ss0, ss1)

    def body(ch, carry):
        base = erow0 + ch * CH
        pltpu.sync_copy(ei3.at[0, pl.ds(base, CH)], srcbuf)
        pltpu.sync_copy(ei3.at[1, pl.ds(base, CH)], dstbuf)
        gh = [None, None]
        sh = [None, None]
        gh[0] = pltpu.async_copy(hs_hbm.at[srcbuf.at[0]], bufs[0], gsems[0])
        for j in range(CH):
            b = j % 2
            gh[b].wait()
            if j + 1 < CH:
                nb = (j + 1) % 2
                if sh[nb] is not None:
                    sh[nb].wait()
                gh[nb] = pltpu.async_copy(
                    hs_hbm.at[srcbuf.at[j + 1]], bufs[nb], gsems[nb])
            sh[b] = pltpu.async_copy(
                bufs[b], acc.at[dstbuf.at[j]], ssems[b], add=True)
        sh[0].wait()
        sh[1].wait()
        return carry

    lax.fori_loop(0, NCH, body, 0)
    plsc.subcore_barrier()
    pltpu.sync_copy(acc.at[pl.ds(r0, ROWS_PER_TILE)],
                    out.at[c, pl.ds(r0, ROWS_PER_TILE)])


BN_ROWS = 2000


def _pre_body(s1_ref, w_ref, g_ref, b_ref, m_ref, v_ref, h_ref):
    h = jnp.dot(s1_ref[...], w_ref[...], preferred_element_type=jnp.float32)
    h = (h - m_ref[...]) * (g_ref[...] * lax.rsqrt(v_ref[...] + 1e-5)) + b_ref[...]
    h_ref[...] = jnp.maximum(h, 0.0)


def _scale_body(h_ref, dp_ref, hs_ref):
    deg = dp_ref[0, :, 0] + dp_ref[1, :, 0] + 1.0
    dinv = lax.rsqrt(jnp.maximum(deg, 1.0))
    hs_ref[...] = h_ref[...] * dinv[:, None]


def _out_body(acc_ref, hs_ref, x0_ref, dp_ref, wg_ref, o_ref):
    deg = dp_ref[0, :, 0] + dp_ref[1, :, 0] + 1.0
    dinv = lax.rsqrt(jnp.maximum(deg, 1.0))
    accsum = acc_ref[0] + acc_ref[1] + hs_ref[...]
    support = (1.0 - ALPHA) * (accsum * dinv[:, None]) + ALPHA * x0_ref[...]
    o_ref[...] = jnp.maximum(
        (1.0 - BETA) * support
        + BETA * jnp.dot(support, wg_ref[...], preferred_element_type=jnp.float32),
        0.0,
    )


def _pre_call(s1, W_pre, bn_gamma, bn_beta, bn_mean, bn_var):
    grid = (N // BN_ROWS,)
    vec = pl.BlockSpec((1, D), lambda i: (0, 0))
    return pl.pallas_call(
        _pre_body,
        grid=grid,
        in_specs=[
            pl.BlockSpec((BN_ROWS, D), lambda i: (i, 0)),
            pl.BlockSpec((D, D), lambda i: (0, 0)),
            vec, vec, vec, vec,
        ],
        out_specs=pl.BlockSpec((BN_ROWS, D), lambda i: (i, 0)),
        out_shape=jax.ShapeDtypeStruct((N, D), jnp.float32),
    )(s1, W_pre, bn_gamma.reshape(1, D), bn_beta.reshape(1, D),
      bn_mean.reshape(1, D), bn_var.reshape(1, D))


def _scale_call(h, degparts):
    grid = (N // BN_ROWS,)
    return pl.pallas_call(
        _scale_body,
        grid=grid,
        in_specs=[
            pl.BlockSpec((BN_ROWS, D), lambda i: (i, 0)),
            pl.BlockSpec((2, BN_ROWS, D), lambda i: (0, i, 0)),
        ],
        out_specs=pl.BlockSpec((BN_ROWS, D), lambda i: (i, 0)),
        out_shape=jax.ShapeDtypeStruct((N, D), jnp.float32),
    )(h, degparts)


def _out_call(accparts, hs, x_0, degparts, W_gin):
    grid = (N // BN_ROWS,)
    return pl.pallas_call(
        _out_body,
        grid=grid,
        in_specs=[
            pl.BlockSpec((2, BN_ROWS, D), lambda i: (0, i, 0)),
            pl.BlockSpec((BN_ROWS, D), lambda i: (i, 0)),
            pl.BlockSpec((BN_ROWS, D), lambda i: (i, 0)),
            pl.BlockSpec((2, BN_ROWS, D), lambda i: (0, i, 0)),
            pl.BlockSpec((D, D), lambda i: (0, 0)),
        ],
        out_specs=pl.BlockSpec((BN_ROWS, D), lambda i: (i, 0)),
        out_shape=jax.ShapeDtypeStruct((N, D), jnp.float32),
    )(accparts, hs, x_0, degparts, W_gin)


def kernel(s0, s1, edge_index, drop_prob, x_0, training,
           W_pre, bn_gamma, bn_beta, bn_mean, bn_var, W_gin):
    ei3 = edge_index.astype(jnp.int32).reshape(2, EROWS, EW)
    ones_ew = jnp.ones((EW, D), jnp.float32)
    zeros_acc = jnp.zeros((NPAD, D), jnp.float32)

    degparts = _deg_call(ei3, ones_ew, zeros_acc)
    h = _pre_call(s1, W_pre, bn_gamma, bn_beta, bn_mean, bn_var)
    hs = _scale_call(h, degparts)
    accparts = _agg_call(ei3, hs, zeros_acc)
    return _out_call(accparts, hs, x_0, degparts, W_gin)


# speedup vs baseline: 29.1879x; 1.0020x over previous
import functools
import math

import jax
import jax.numpy as jnp
from jax import lax
from jax.experimental import pallas as pl
from jax.experimental.pallas import tpu as pltpu
from jax.experimental.pallas import tpu_sc as plsc

N = 10000
E = 320000
D = 128
ALPHA = 0.1
BETA = float(math.log(0.5 / 1 + 1.0))

NPAD = 10240
ROWS_PER_TILE = NPAD // 16
EW = 125
EROWS = E // EW
WORKERS = 32
ROWS_PER_WORKER = EROWS // WORKERS
CH = 8
NCH = ROWS_PER_WORKER // CH

_MESH = plsc.VectorSubcoreMesh(core_axis_name="c", subcore_axis_name="s")


@functools.partial(
    pl.kernel,
    mesh=_MESH,
    out_type=jax.ShapeDtypeStruct((2, NPAD, D), jnp.float32),
    scratch_types=[
        pltpu.VMEM((ROWS_PER_WORKER, EW), jnp.int32),
        pltpu.VMEM((EW, D), jnp.float32),
        pltpu.VMEM_SHARED((NPAD, D), jnp.float32),
        pltpu.SemaphoreType.DMA,
    ],
)
def _deg_call(ei3, ones_hbm, zeros_hbm, out, dstbuf, onesbuf, acc, dsem):
    c = lax.axis_index("c")
    s = lax.axis_index("s")
    wid = c * 16 + s
    r0 = s * ROWS_PER_TILE
    pltpu.sync_copy(zeros_hbm.at[pl.ds(r0, ROWS_PER_TILE)],
                    acc.at[pl.ds(r0, ROWS_PER_TILE)])
    pltpu.sync_copy(ones_hbm, onesbuf)
    erow0 = wid * ROWS_PER_WORKER
    pltpu.sync_copy(ei3.at[1, pl.ds(erow0, ROWS_PER_WORKER)], dstbuf)
    plsc.subcore_barrier()

    def body(ch, carry):
        hs_ = [pltpu.async_copy(onesbuf, acc.at[dstbuf.at[ch * CH + j]],
                                dsem, add=True)
               for j in range(CH)]
        for h_ in hs_:
            h_.wait()
        return carry

    lax.fori_loop(0, NCH, body, 0)
    plsc.subcore_barrier()
    pltpu.sync_copy(acc.at[pl.ds(r0, ROWS_PER_TILE)],
                    out.at[c, pl.ds(r0, ROWS_PER_TILE)])


@functools.partial(
    pl.kernel,
    mesh=_MESH,
    out_type=jax.ShapeDtypeStruct((2, NPAD, D), jnp.float32),
    scratch_types=[
        pltpu.VMEM((CH, EW), jnp.int32),
        pltpu.VMEM((CH, EW), jnp.int32),
        pltpu.VMEM((EW, D), jnp.float32),
        pltpu.VMEM((EW, D), jnp.float32),
        pltpu.VMEM_SHARED((NPAD, D), jnp.float32),
        pltpu.SemaphoreType.DMA,
        pltpu.SemaphoreType.DMA,
        pltpu.SemaphoreType.DMA,
        pltpu.SemaphoreType.DMA,
    ],
)
def _agg_call(ei3, hs_hbm, zeros_hbm, out,
              srcbuf, dstbuf, rowbuf0, rowbuf1, acc, gs0, gs1, ss0, ss1):
    c = lax.axis_index("c")
    s = lax.axis_index("s")
    wid = c * 16 + s
    r0 = s * ROWS_PER_TILE
    pltpu.sync_copy(zeros_hbm.at[pl.ds(r0, ROWS_PER_TILE)],
                    acc.at[pl.ds(r0, ROWS_PER_TILE)])
    plsc.subcore_barrier()
    erow0 = wid * ROWS_PER_WORKER
    bufs = (rowbuf0, rowbuf1)
    gsems = (gs0, gs1)
    ssems = (ss0, ss1)

    def body(ch, carry):
        base = erow0 + ch * CH
        pltpu.sync_copy(ei3.at[0, pl.ds(base, CH)], srcbuf)
        pltpu.sync_copy(ei3.at[1, pl.ds(base, CH)], dstbuf)
        gh = [None, None]
        sh = [None, None]
        gh[0] = pltpu.async_copy(hs_hbm.at[srcbuf.at[0]], bufs[0], gsems[0])
        for j in range(CH):
            b = j % 2
            gh[b].wait()
            if j + 1 < CH:
                nb = (j + 1) % 2
                if sh[nb] is not None:
                    sh[nb].wait()
                gh[nb] = pltpu.async_copy(
                    hs_hbm.at[srcbuf.at[j + 1]], bufs[nb], gsems[nb])
            sh[b] = pltpu.async_copy(
                bufs[b], acc.at[dstbuf.at[j]], ssems[b], add=True)
        sh[0].wait()
        sh[1].wait()
        return carry

    lax.fori_loop(0, NCH, body, 0)
    plsc.subcore_barrier()
    pltpu.sync_copy(acc.at[pl.ds(r0, ROWS_PER_TILE)],
                    out.at[c, pl.ds(r0, ROWS_PER_TILE)])


BN_ROWS = 2000


def _pre_body(s1_ref, w_ref, g_ref, b_ref, m_ref, v_ref, h_ref):
    h = jnp.dot(s1_ref[...], w_ref[...], preferred_element_type=jnp.float32)
    h = (h - m_ref[...]) * (g_ref[...] * lax.rsqrt(v_ref[...] + 1e-5)) + b_ref[...]
    h_ref[...] = jnp.maximum(h, 0.0)


def _scale_body(h_ref, dp_ref, hs_ref, dinv_ref):
    deg = dp_ref[0, :, 0] + dp_ref[1, :, 0] + 1.0
    dinv = lax.rsqrt(jnp.maximum(deg, 1.0))
    hs_ref[...] = h_ref[...] * dinv[:, None]
    dinv_ref[...] = dinv[:, None]


def _out_body(acc_ref, hs_ref, x0_ref, dinv_ref, wg_ref, o_ref):
    dinv = dinv_ref[...]
    accsum = acc_ref[0] + acc_ref[1] + hs_ref[...]
    support = (1.0 - ALPHA) * (accsum * dinv) + ALPHA * x0_ref[...]
    o_ref[...] = jnp.maximum(
        (1.0 - BETA) * support
        + BETA * jnp.dot(support, wg_ref[...], preferred_element_type=jnp.float32),
        0.0,
    )


def _pre_call(s1, W_pre, bn_gamma, bn_beta, bn_mean, bn_var):
    grid = (N // BN_ROWS,)
    vec = pl.BlockSpec((1, D), lambda i: (0, 0))
    return pl.pallas_call(
        _pre_body,
        grid=grid,
        in_specs=[
            pl.BlockSpec((BN_ROWS, D), lambda i: (i, 0)),
            pl.BlockSpec((D, D), lambda i: (0, 0)),
            vec, vec, vec, vec,
        ],
        out_specs=pl.BlockSpec((BN_ROWS, D), lambda i: (i, 0)),
        out_shape=jax.ShapeDtypeStruct((N, D), jnp.float32),
    )(s1, W_pre, bn_gamma.reshape(1, D), bn_beta.reshape(1, D),
      bn_mean.reshape(1, D), bn_var.reshape(1, D))


def _scale_call(h, degparts):
    grid = (N // BN_ROWS,)
    return pl.pallas_call(
        _scale_body,
        grid=grid,
        in_specs=[
            pl.BlockSpec((BN_ROWS, D), lambda i: (i, 0)),
            pl.BlockSpec((2, BN_ROWS, D), lambda i: (0, i, 0)),
        ],
        out_specs=[pl.BlockSpec((BN_ROWS, D), lambda i: (i, 0)),
                   pl.BlockSpec((BN_ROWS, 1), lambda i: (i, 0))],
        out_shape=[jax.ShapeDtypeStruct((N, D), jnp.float32),
                   jax.ShapeDtypeStruct((N, 1), jnp.float32)],
    )(h, degparts)


def _out_call(accparts, hs, x_0, dinv, W_gin):
    grid = (N // BN_ROWS,)
    return pl.pallas_call(
        _out_body,
        grid=grid,
        in_specs=[
            pl.BlockSpec((2, BN_ROWS, D), lambda i: (0, i, 0)),
            pl.BlockSpec((BN_ROWS, D), lambda i: (i, 0)),
            pl.BlockSpec((BN_ROWS, D), lambda i: (i, 0)),
            pl.BlockSpec((BN_ROWS, 1), lambda i: (i, 0)),
            pl.BlockSpec((D, D), lambda i: (0, 0)),
        ],
        out_specs=pl.BlockSpec((BN_ROWS, D), lambda i: (i, 0)),
        out_shape=jax.ShapeDtypeStruct((N, D), jnp.float32),
    )(accparts, hs, x_0, dinv, W_gin)


def kernel(s0, s1, edge_index, drop_prob, x_0, training,
           W_pre, bn_gamma, bn_beta, bn_mean, bn_var, W_gin):
    ei3 = edge_index.astype(jnp.int32).reshape(2, EROWS, EW)
    ones_ew = jnp.ones((EW, D), jnp.float32)
    zeros_acc = jnp.zeros((NPAD, D), jnp.float32)

    degparts = _deg_call(ei3, ones_ew, zeros_acc)
    h = _pre_call(s1, W_pre, bn_gamma, bn_beta, bn_mean, bn_var)
    hs, dinv = _scale_call(h, degparts)
    accparts = _agg_call(ei3, hs, zeros_acc)
    return _out_call(accparts, hs, x_0, dinv, W_gin)
